# baseline (speedup 1.0000x reference)
import functools

import jax
import jax.numpy as jnp
from jax import lax
from jax.experimental import pallas as pl
from jax.experimental.pallas import tpu as pltpu
from jax.experimental.pallas import tpu_sc as plsc

K = 16
M1, M2 = 20, 10
NC, NS = 2, 16
NW = NC * NS
CB = 8


def _fm_body(idx1_hbm, idx2_hbm, v1_hbm, v2_hbm, dum1_hbm, dum2_hbm, out_hbm,
             idx1_w, idx2_w,
             rows1_a, rows1_b, rows2_a, rows2_b,
             out_v,
             sem1_a, sem1_b, sem2_a, sem2_b,
             *, batch):
    per_w = batch // NW
    nblk = per_w // CB
    wid = lax.axis_index("s") * NC + lax.axis_index("c")
    base = wid * per_w
    rows1_v = (rows1_a, rows1_b)
    rows2_v = (rows2_a, rows2_b)
    sem1 = (sem1_a, sem1_b)
    sem2 = (sem2_a, sem2_b)

    lanes = lax.iota(jnp.int32, K)
    zeros = jnp.zeros((K,), jnp.float32)
    lane0 = lanes == 0

    pltpu.sync_copy(idx1_hbm.at[pl.ds(base * M1, per_w * M1)], idx1_w)
    pltpu.sync_copy(idx2_hbm.at[pl.ds(base * M2, per_w * M2)], idx2_w)

    def fire_rows(tab_hbm, idx_w, rows_v, sem, t, m):
        def grp(g, _):
            iv = idx_w[pl.ds(t * CB * m + g * K, K)]
            for l in range(K):
                r = iv[l]
                pltpu.make_async_copy(
                    tab_hbm.at[pl.ds(r, 1)],
                    rows_v.at[pl.ds(g * K + l, 1)], sem).start()
            return _
        lax.fori_loop(0, CB * m // K, grp, None)

    def start_gathers(t, ph):
        fire_rows(v1_hbm, idx1_w, rows1_v[ph], sem1[ph], t, M1)
        fire_rows(v2_hbm, idx2_w, rows2_v[ph], sem2[ph], t, M2)

    def wait_gathers(ph):
        pltpu.make_async_copy(dum1_hbm, rows1_v[ph], sem1[ph]).wait()
        pltpu.make_async_copy(dum2_hbm, rows2_v[ph], sem2[ph]).wait()

    def zero_pass(idx_w, rows_v, t, m):
        def grp(g, _):
            ivec = idx_w[pl.ds(t * CB * m + g * K, K)]
            mz = ivec == 0
            rowids = g * K + lanes
            for k in range(K):
                plsc.store_scatter(
                    rows_v, [rowids, jnp.full((K,), k, jnp.int32)],
                    zeros, mask=mz)
            return _
        lax.fori_loop(0, CB * m // K, grp, None)

    NG1 = CB * M1 // K
    NG2 = CB * M2 // K

    def fire_grp_dyn(tab_hbm, idx_w, rows_v, sem, t, m, g):
        iv = idx_w[pl.ds(t * CB * m + g * K, K)]
        for l in range(K):
            r = iv[l]
            pltpu.make_async_copy(
                tab_hbm.at[pl.ds(r, 1)],
                rows_v.at[pl.ds(g * K + l, 1)], sem).start()

    def compute(t, ph, fire_next):
        zero_pass(idx1_w, rows1_v[ph], t, M1)
        zero_pass(idx2_w, rows2_v[ph], t, M2)
        r1 = rows1_v[ph]
        r2 = rows2_v[ph]

        def elem(i, _):
            for gg in range(2):
                g = i * 2 + gg

                @pl.when(jnp.logical_and(fire_next, g < NG1))
                def _():
                    fire_grp_dyn(v1_hbm, idx1_w, rows1_v[1 - ph],
                                 sem1[1 - ph], t + 1, M1, g)

                @pl.when(jnp.logical_and(
                    fire_next, jnp.logical_and(g >= NG1, g < NG1 + NG2)))
                def _():
                    fire_grp_dyn(v2_hbm, idx2_w, rows2_v[1 - ph],
                                 sem2[1 - ph], t + 1, M2, g - NG1)

            s = jnp.zeros((K,), jnp.float32)
            q = jnp.zeros((K,), jnp.float32)
            for j in range(M1):
                r = r1[i * M1 + j]
                s = s + r
                q = q + r * r
            for j in range(M2):
                r = r2[i * M2 + j]
                s = s + r
                q = q + r * r
            red = 0.5 * jnp.sum(s * s - q)
            plsc.store_scatter(out_v, [jnp.full((K,), t * CB + i, jnp.int32)],
                               jnp.broadcast_to(red, (K,)), mask=lane0)
            return _

        lax.fori_loop(0, CB, elem, None)

    start_gathers(0, 0)

    def pair(p, _):
        for ph in range(2):
            t = 2 * p + ph

            @pl.when(t < nblk)
            def _():
                wait_gathers(ph)
                compute(t, ph, t + 1 < nblk)
        return _

    lax.fori_loop(0, (nblk + 1) // 2, pair, None)
    pltpu.sync_copy(out_v, out_hbm.at[pl.ds(base, per_w)])


def kernel(idx1, idx2, V1, V2):
    batch = idx1.shape[0]
    per_w = batch // NW
    mesh = plsc.VectorSubcoreMesh(
        core_axis_name="c", subcore_axis_name="s",
        num_cores=NC, num_subcores=NS)
    run = pl.kernel(
        functools.partial(_fm_body, batch=batch),
        out_type=jax.ShapeDtypeStruct((batch,), jnp.float32),
        mesh=mesh,
        scratch_types=[
            pltpu.VMEM((per_w * M1,), jnp.int32),
            pltpu.VMEM((per_w * M2,), jnp.int32),
            pltpu.VMEM((CB * M1, K), jnp.float32),
            pltpu.VMEM((CB * M1, K), jnp.float32),
            pltpu.VMEM((CB * M2, K), jnp.float32),
            pltpu.VMEM((CB * M2, K), jnp.float32),
            pltpu.VMEM((per_w,), jnp.float32),
            pltpu.SemaphoreType.DMA,
            pltpu.SemaphoreType.DMA,
            pltpu.SemaphoreType.DMA,
            pltpu.SemaphoreType.DMA,
        ],
        compiler_params=pltpu.CompilerParams(
            needs_layout_passes=False, use_tc_tiling_on_sc=True),
    )
    return run(idx1.reshape(-1), idx2.reshape(-1), V1, V2,
               jnp.zeros((CB * M1, K), jnp.float32),
               jnp.zeros((CB * M2, K), jnp.float32))
